# Initial kernel scaffold; baseline (speedup 1.0000x reference)
import functools

import jax
import jax.numpy as jnp
from jax import lax
from jax.experimental import pallas as pl
from jax.experimental.pallas import tpu as pltpu
from jax.experimental.pallas import tpu_sc as plsc

B, L = 4096, 200
N = B * L
EDIM = 16
OUT_F = 32

NC, NS = 2, 16
NW = NC * NS
PER_W = N // NW
CHUNK = 1600
NCHUNK = PER_W // CHUNK


def _gather_sc(table, idx):
    mesh = plsc.VectorSubcoreMesh(core_axis_name="c", subcore_axis_name="s")

    @functools.partial(
        pl.kernel,
        mesh=mesh,
        out_type=jax.ShapeDtypeStruct((N, EDIM), jnp.float32),
        scratch_types=[
            pltpu.VMEM((CHUNK,), jnp.int32),
            pltpu.VMEM((CHUNK, EDIM), jnp.float32),
            pltpu.SemaphoreType.DMA,
        ],
    )
    def k(table_hbm, idx_hbm, out_hbm, idx_v, rows_v, sem):
        wid = lax.axis_index("s") * NC + lax.axis_index("c")
        base = wid * PER_W

        @pl.loop(0, NCHUNK)
        def _(c):
            off = base + c * CHUNK
            pltpu.sync_copy(idx_hbm.at[pl.ds(off, CHUNK)], idx_v)
            pltpu.async_copy(table_hbm.at[idx_v], rows_v, sem).wait()
            pltpu.sync_copy(rows_v, out_hbm.at[pl.ds(off, CHUNK)])

    return k(table, idx)


T = 2048
NT = N // T


def _dense_body(lng_ref, lat_ref, g_ref, s_ref, wg_ref, ws_ref, c_ref, o_ref):
    dn = (((1,), (0,)), ((), ()))
    acc = lax.dot_general(g_ref[...], wg_ref[...], dn,
                          preferred_element_type=jnp.float32)
    acc += lax.dot_general(s_ref[...], ws_ref[...], dn,
                           preferred_element_type=jnp.float32)
    acc += lng_ref[...] * c_ref[0:1, :]
    acc += lat_ref[...] * c_ref[1:2, :]
    acc += c_ref[2:3, :]
    o_ref[...] = jnp.tanh(acc)


def _dense_tc(lng, lat, grids, sdne, wg, ws, consts):
    return pl.pallas_call(
        _dense_body,
        grid=(NT,),
        in_specs=[
            pl.BlockSpec((T, 1), lambda i: (i, 0)),
            pl.BlockSpec((T, 1), lambda i: (i, 0)),
            pl.BlockSpec((T, EDIM), lambda i: (i, 0)),
            pl.BlockSpec((T, EDIM), lambda i: (i, 0)),
            pl.BlockSpec((EDIM, OUT_F), lambda i: (0, 0)),
            pl.BlockSpec((EDIM, OUT_F), lambda i: (0, 0)),
            pl.BlockSpec((8, OUT_F), lambda i: (0, 0)),
        ],
        out_specs=pl.BlockSpec((T, OUT_F), lambda i: (i, 0)),
        out_shape=jax.ShapeDtypeStruct((N, OUT_F), jnp.float32),
    )(lng, lat, grids, sdne, wg, ws, consts)


@jax.jit
def kernel(lngs, lats, grid_id, SDNE_embedding, emb_table, W, b):
    idx = grid_id.reshape(N).astype(jnp.int32)
    grids = _gather_sc(emb_table, idx)
    lng = lngs.reshape(N, 1)
    lat = lats.reshape(N, 1)
    sdne = SDNE_embedding.reshape(N, EDIM)
    wg = W[2:2 + EDIM]
    ws = W[2 + EDIM:]
    consts = jnp.concatenate(
        [W[0:1], W[1:2], b[None, :], jnp.zeros((5, OUT_F), W.dtype)], axis=0)
    out = _dense_tc(lng, lat, grids, sdne, wg, ws, consts)
    return out.reshape(B, L, OUT_F)


# speedup vs baseline: 1.8523x; 1.8523x over previous
import functools

import jax
import jax.numpy as jnp
from jax import lax
from jax.experimental import pallas as pl
from jax.experimental.pallas import tpu as pltpu
from jax.experimental.pallas import tpu_sc as plsc

B, L = 4096, 200
N = B * L
EDIM = 16
OUT_F = 32

NC, NS = 2, 16
NW = NC * NS
PER_W = N // NW
CHUNK = 1600
NCHUNK = PER_W // CHUNK


def _gather_sc(table, idx):
    mesh = plsc.VectorSubcoreMesh(core_axis_name="c", subcore_axis_name="s")

    @functools.partial(
        pl.kernel,
        mesh=mesh,
        compiler_params=pltpu.CompilerParams(use_tc_tiling_on_sc=False),
        out_type=jax.ShapeDtypeStruct((N, EDIM), jnp.float32),
        scratch_types=[
            pltpu.VMEM((CHUNK,), jnp.int32),
            pltpu.VMEM((CHUNK, EDIM), jnp.float32),
            pltpu.SemaphoreType.DMA,
        ],
    )
    def k(table_hbm, idx_hbm, out_hbm, idx_v, rows_v, sem):
        wid = lax.axis_index("s") * NC + lax.axis_index("c")
        base = wid * PER_W

        @pl.loop(0, NCHUNK)
        def _(c):
            off = base + c * CHUNK
            pltpu.sync_copy(idx_hbm.at[pl.ds(off, CHUNK)], idx_v)
            pltpu.async_copy(table_hbm.at[idx_v], rows_v, sem).wait()
            pltpu.sync_copy(rows_v, out_hbm.at[pl.ds(off, CHUNK)])

    return k(table, idx)


T = 2048
NT = N // T


def _dense_body(lng_ref, lat_ref, g_ref, s_ref, wg_ref, ws_ref, c_ref, o_ref):
    dn = (((1,), (0,)), ((), ()))
    acc = lax.dot_general(g_ref[...], wg_ref[...], dn,
                          preferred_element_type=jnp.float32)
    acc += lax.dot_general(s_ref[...], ws_ref[...], dn,
                           preferred_element_type=jnp.float32)
    acc += lng_ref[...] * c_ref[0:1, :]
    acc += lat_ref[...] * c_ref[1:2, :]
    acc += c_ref[2:3, :]
    o_ref[...] = jnp.tanh(acc)


def _dense_tc(lng, lat, grids, sdne, wg, ws, consts):
    return pl.pallas_call(
        _dense_body,
        grid=(NT,),
        in_specs=[
            pl.BlockSpec((T, 1), lambda i: (i, 0)),
            pl.BlockSpec((T, 1), lambda i: (i, 0)),
            pl.BlockSpec((T, EDIM), lambda i: (i, 0)),
            pl.BlockSpec((T, EDIM), lambda i: (i, 0)),
            pl.BlockSpec((EDIM, OUT_F), lambda i: (0, 0)),
            pl.BlockSpec((EDIM, OUT_F), lambda i: (0, 0)),
            pl.BlockSpec((8, OUT_F), lambda i: (0, 0)),
        ],
        out_specs=pl.BlockSpec((T, OUT_F), lambda i: (i, 0)),
        out_shape=jax.ShapeDtypeStruct((N, OUT_F), jnp.float32),
    )(lng, lat, grids, sdne, wg, ws, consts)


@jax.jit
def kernel(lngs, lats, grid_id, SDNE_embedding, emb_table, W, b):
    idx = grid_id.reshape(N).astype(jnp.int32)
    grids = _gather_sc(emb_table, idx)
    lng = lngs.reshape(N, 1)
    lat = lats.reshape(N, 1)
    sdne = SDNE_embedding.reshape(N, EDIM)
    wg = W[2:2 + EDIM]
    ws = W[2 + EDIM:]
    consts = jnp.concatenate(
        [W[0:1], W[1:2], b[None, :], jnp.zeros((5, OUT_F), W.dtype)], axis=0)
    out = _dense_tc(lng, lat, grids, sdne, wg, ws, consts)
    return out.reshape(B, L, OUT_F)


# speedup vs baseline: 7.9136x; 4.2722x over previous
import functools

import jax
import jax.numpy as jnp
from jax import lax
from jax.experimental import pallas as pl
from jax.experimental.pallas import tpu as pltpu
from jax.experimental.pallas import tpu_sc as plsc

B, L = 4096, 200
N = B * L
EDIM = 16
OUT_F = 32

NC, NS = 2, 16
NW = NC * NS
PER_W = N // NW
CHUNK = 1600
NCHUNK = PER_W // CHUNK


def _gather_sc(table, idx):
    mesh = plsc.VectorSubcoreMesh(core_axis_name="c", subcore_axis_name="s")

    @functools.partial(
        pl.kernel,
        mesh=mesh,
        compiler_params=pltpu.CompilerParams(use_tc_tiling_on_sc=False),
        out_type=jax.ShapeDtypeStruct((N, EDIM), jnp.float32),
        scratch_types=[
            pltpu.VMEM((CHUNK,), jnp.int32),
            pltpu.VMEM((CHUNK, EDIM), jnp.float32),
            pltpu.SemaphoreType.DMA,
        ],
    )
    def k(table_hbm, idx_hbm, out_hbm, idx_v, rows_v, sem):
        wid = lax.axis_index("s") * NC + lax.axis_index("c")
        base = wid * PER_W

        @pl.loop(0, NCHUNK)
        def _(c):
            off = base + c * CHUNK
            pltpu.sync_copy(idx_hbm.at[pl.ds(off, CHUNK)], idx_v)
            pltpu.async_copy(table_hbm.at[idx_v], rows_v, sem).wait()
            pltpu.sync_copy(rows_v, out_hbm.at[pl.ds(off, CHUNK)])

    return k(table, idx)


PK = B // 8


def _dense_body(lng_ref, lat_ref, s_ref, g_ref, wpack_ref, wst_ref, c_ref,
                o_ref):
    f32 = jnp.float32
    S = s_ref[0]
    acc = lax.dot_general(wst_ref[...], S, (((1,), (0,)), ((), ())),
                          preferred_element_type=f32)
    R = lax.dot_general(g_ref[...], wpack_ref[...], (((1,), (0,)), ((), ())),
                        preferred_element_type=f32)
    RT = jnp.transpose(R, (1, 0))
    acc += jnp.concatenate(
        [RT[OUT_F * s:OUT_F * (s + 1), :] for s in range(8)], axis=1)
    acc += c_ref[:, 0:1] * lng_ref[0]
    acc += c_ref[:, 1:2] * lat_ref[0]
    acc += c_ref[:, 2:3]
    o_ref[...] = jnp.tanh(acc).reshape(1, OUT_F, B)


def _dense_tc(lngT, latT, sdneT, grids2, wpack, wst, consts):
    return pl.pallas_call(
        _dense_body,
        grid=(L,),
        in_specs=[
            pl.BlockSpec((1, 1, B), lambda i: (i, 0, 0)),
            pl.BlockSpec((1, 1, B), lambda i: (i, 0, 0)),
            pl.BlockSpec((1, EDIM, B), lambda i: (i, 0, 0)),
            pl.BlockSpec((PK, 128), lambda i: (i, 0)),
            pl.BlockSpec((128, 8 * OUT_F), lambda i: (0, 0)),
            pl.BlockSpec((OUT_F, EDIM), lambda i: (0, 0)),
            pl.BlockSpec((OUT_F, 128), lambda i: (0, 0)),
        ],
        out_specs=pl.BlockSpec((1, OUT_F, B), lambda i: (i, 0, 0)),
        out_shape=jax.ShapeDtypeStruct((L, OUT_F, B), jnp.float32),
    )(lngT, latT, sdneT, grids2, wpack, wst, consts)


@jax.jit
def kernel(lngs, lats, grid_id, SDNE_embedding, emb_table, W, b):
    idx = jnp.transpose(grid_id, (1, 0)).reshape(L, 8, PK)
    idx = jnp.transpose(idx, (0, 2, 1)).reshape(N).astype(jnp.int32)
    lngT = jnp.transpose(lngs, (1, 0)).reshape(L, 1, B)
    latT = jnp.transpose(lats, (1, 0)).reshape(L, 1, B)
    sdneT = jnp.transpose(SDNE_embedding, (1, 2, 0))

    grids = _gather_sc(emb_table, idx)
    grids2 = grids.reshape(N // 8, 128)

    wg = W[2:2 + EDIM]
    wpack = jax.scipy.linalg.block_diag(*([wg] * 8))
    wst = jnp.transpose(W[2 + EDIM:], (1, 0))
    consts = jnp.zeros((OUT_F, 128), jnp.float32)
    consts = consts.at[:, 0].set(W[0]).at[:, 1].set(W[1]).at[:, 2].set(b)

    outT = _dense_tc(lngT, latT, sdneT, grids2, wpack, wst, consts)
    return jnp.transpose(outT, (2, 0, 1))


# speedup vs baseline: 8.3832x; 1.0593x over previous
import functools

import jax
import jax.numpy as jnp
from jax import lax
from jax.experimental import pallas as pl
from jax.experimental.pallas import tpu as pltpu
from jax.experimental.pallas import tpu_sc as plsc

B, L = 4096, 200
N = B * L
EDIM = 16
OUT_F = 32

NC, NS = 2, 16
NW = NC * NS
PER_W = N // NW
CHUNK = 3200
NCHUNK = PER_W // CHUNK


def _gather_sc(table, idx):
    mesh = plsc.VectorSubcoreMesh(core_axis_name="c", subcore_axis_name="s")

    @functools.partial(
        pl.kernel,
        mesh=mesh,
        compiler_params=pltpu.CompilerParams(use_tc_tiling_on_sc=False),
        out_type=jax.ShapeDtypeStruct((N, EDIM), jnp.float32),
        scratch_types=[
            pltpu.VMEM((CHUNK,), jnp.int32),
            pltpu.VMEM((CHUNK,), jnp.int32),
            pltpu.VMEM((CHUNK, EDIM), jnp.float32),
            pltpu.VMEM((CHUNK, EDIM), jnp.float32),
            pltpu.SemaphoreType.DMA,
        ],
    )
    def k(table_hbm, idx_hbm, out_hbm, idx_v0, idx_v1, rows_v0, rows_v1,
          gsem):
        wid = lax.axis_index("s") * NC + lax.axis_index("c")
        base = wid * PER_W
        idx_bufs = (idx_v0, idx_v1)
        row_bufs = (rows_v0, rows_v1)

        def off(c):
            return base + c * CHUNK

        pltpu.sync_copy(idx_hbm.at[pl.ds(off(0), CHUNK)], idx_bufs[0])
        gathers = [pltpu.async_copy(table_hbm.at[idx_bufs[0]], row_bufs[0],
                                    gsem)]
        if NCHUNK > 1:
            pltpu.sync_copy(idx_hbm.at[pl.ds(off(1), CHUNK)], idx_bufs[1])
        for c in range(NCHUNK):
            gathers[c].wait()
            if c + 1 < NCHUNK:
                gathers.append(
                    pltpu.async_copy(table_hbm.at[idx_bufs[(c + 1) % 2]],
                                     row_bufs[(c + 1) % 2], gsem))
            pltpu.sync_copy(row_bufs[c % 2],
                            out_hbm.at[pl.ds(off(c), CHUNK)])
            if c + 2 < NCHUNK:
                pltpu.sync_copy(idx_hbm.at[pl.ds(off(c + 2), CHUNK)],
                                idx_bufs[c % 2])

    return k(table, idx)


PK = B // 8


def _dense_body(lng_ref, lat_ref, s_ref, g_ref, wpack_ref, wst_ref, c_ref,
                o_ref):
    f32 = jnp.float32
    S = s_ref[0]
    acc = lax.dot_general(wst_ref[...], S, (((1,), (0,)), ((), ())),
                          preferred_element_type=f32)
    R = lax.dot_general(g_ref[...], wpack_ref[...], (((1,), (0,)), ((), ())),
                        preferred_element_type=f32)
    RT = jnp.transpose(R, (1, 0))
    acc += jnp.concatenate(
        [RT[OUT_F * s:OUT_F * (s + 1), :] for s in range(8)], axis=1)
    acc += c_ref[:, 0:1] * lng_ref[0]
    acc += c_ref[:, 1:2] * lat_ref[0]
    acc += c_ref[:, 2:3]
    o_ref[...] = jnp.tanh(acc).reshape(1, OUT_F, B)


def _dense_tc(lngT, latT, sdneT, grids2, wpack, wst, consts):
    return pl.pallas_call(
        _dense_body,
        grid=(L,),
        in_specs=[
            pl.BlockSpec((1, 1, B), lambda i: (i, 0, 0)),
            pl.BlockSpec((1, 1, B), lambda i: (i, 0, 0)),
            pl.BlockSpec((1, EDIM, B), lambda i: (i, 0, 0)),
            pl.BlockSpec((PK, 128), lambda i: (i, 0)),
            pl.BlockSpec((128, 8 * OUT_F), lambda i: (0, 0)),
            pl.BlockSpec((OUT_F, EDIM), lambda i: (0, 0)),
            pl.BlockSpec((OUT_F, 128), lambda i: (0, 0)),
        ],
        out_specs=pl.BlockSpec((1, OUT_F, B), lambda i: (i, 0, 0)),
        out_shape=jax.ShapeDtypeStruct((L, OUT_F, B), jnp.float32),
    )(lngT, latT, sdneT, grids2, wpack, wst, consts)


@jax.jit
def kernel(lngs, lats, grid_id, SDNE_embedding, emb_table, W, b):
    idx = jnp.transpose(grid_id, (1, 0)).reshape(L, 8, PK)
    idx = jnp.transpose(idx, (0, 2, 1)).reshape(N).astype(jnp.int32)
    lngT = jnp.transpose(lngs, (1, 0)).reshape(L, 1, B)
    latT = jnp.transpose(lats, (1, 0)).reshape(L, 1, B)
    sdneT = jnp.transpose(SDNE_embedding, (1, 2, 0))

    grids = _gather_sc(emb_table, idx)
    grids2 = grids.reshape(N // 8, 128)

    wg = W[2:2 + EDIM]
    wpack = jax.scipy.linalg.block_diag(*([wg] * 8))
    wst = jnp.transpose(W[2 + EDIM:], (1, 0))
    consts = jnp.zeros((OUT_F, 128), jnp.float32)
    consts = consts.at[:, 0].set(W[0]).at[:, 1].set(W[1]).at[:, 2].set(b)

    outT = _dense_tc(lngT, latT, sdneT, grids2, wpack, wst, consts)
    return jnp.transpose(outT, (2, 0, 1))


# speedup vs baseline: 8.3891x; 1.0007x over previous
import functools

import jax
import jax.numpy as jnp
from jax import lax
from jax.experimental import pallas as pl
from jax.experimental.pallas import tpu as pltpu
from jax.experimental.pallas import tpu_sc as plsc

B, L = 4096, 200
N = B * L
EDIM = 16
OUT_F = 32
VOCAB = 65536

NC, NS = 2, 16
NW = NC * NS
PER_W = N // NW
CHUNK = 3200
NCHUNK = PER_W // CHUNK


def _gather_sc(table, idx):
    mesh = plsc.VectorSubcoreMesh(core_axis_name="c", subcore_axis_name="s")

    @functools.partial(
        pl.kernel,
        mesh=mesh,
        compiler_params=pltpu.CompilerParams(use_tc_tiling_on_sc=False),
        out_type=jax.ShapeDtypeStruct((N, EDIM), jnp.float32),
        scratch_types=[
            pltpu.VMEM((CHUNK,), jnp.int32),
            pltpu.VMEM((CHUNK,), jnp.int32),
            pltpu.VMEM((CHUNK, EDIM), jnp.float32),
            pltpu.VMEM((CHUNK, EDIM), jnp.float32),
            pltpu.SemaphoreType.DMA,
        ],
    )
    def k(table_hbm, idx_hbm, out_hbm, idx_v0, idx_v1, rows_v0, rows_v1,
          gsem):
        wid = lax.axis_index("s") * NC + lax.axis_index("c")
        base = wid * PER_W
        idx_bufs = (idx_v0, idx_v1)
        row_bufs = (rows_v0, rows_v1)

        def off(c):
            return base + c * CHUNK

        pltpu.sync_copy(idx_hbm.at[pl.ds(off(0), CHUNK)], idx_bufs[0])
        gathers = [pltpu.async_copy(table_hbm.at[idx_bufs[0]], row_bufs[0],
                                    gsem)]
        if NCHUNK > 1:
            pltpu.sync_copy(idx_hbm.at[pl.ds(off(1), CHUNK)], idx_bufs[1])
        for c in range(NCHUNK):
            gathers[c].wait()
            if c + 1 < NCHUNK:
                gathers.append(
                    pltpu.async_copy(table_hbm.at[idx_bufs[(c + 1) % 2]],
                                     row_bufs[(c + 1) % 2], gsem))
            pltpu.sync_copy(row_bufs[c % 2],
                            out_hbm.at[pl.ds(off(c), CHUNK)])
            if c + 2 < NCHUNK:
                pltpu.sync_copy(idx_hbm.at[pl.ds(off(c + 2), CHUNK)],
                                idx_bufs[c % 2])

    return k(table, idx)


PK = B // 8


def _dense_body(lng_ref, lat_ref, s_ref, g_ref, wpack_ref, wst_ref, c_ref,
                o_ref):
    f32 = jnp.float32
    S = s_ref[0]
    acc = lax.dot_general(wst_ref[...], S, (((1,), (0,)), ((), ())),
                          preferred_element_type=f32)
    R = lax.dot_general(g_ref[...], wpack_ref[...], (((1,), (0,)), ((), ())),
                        preferred_element_type=f32)
    RT = jnp.transpose(R, (1, 0))
    acc += jnp.concatenate(
        [RT[OUT_F * s:OUT_F * (s + 1), :] for s in range(8)], axis=1)
    acc += c_ref[:, 0:1] * lng_ref[0]
    acc += c_ref[:, 1:2] * lat_ref[0]
    acc += c_ref[:, 2:3]
    o_ref[...] = jnp.tanh(acc).reshape(1, OUT_F, B)


def _dense_tc(lngT, latT, sdneT, grids2, wpack, wst, consts):
    return pl.pallas_call(
        _dense_body,
        grid=(L,),
        in_specs=[
            pl.BlockSpec((1, 1, B), lambda i: (i, 0, 0)),
            pl.BlockSpec((1, 1, B), lambda i: (i, 0, 0)),
            pl.BlockSpec((1, EDIM, B), lambda i: (i, 0, 0)),
            pl.BlockSpec((PK, 128), lambda i: (i, 0)),
            pl.BlockSpec((128, 8 * OUT_F), lambda i: (0, 0)),
            pl.BlockSpec((OUT_F, EDIM), lambda i: (0, 0)),
            pl.BlockSpec((OUT_F, 128), lambda i: (0, 0)),
        ],
        out_specs=pl.BlockSpec((1, OUT_F, B), lambda i: (i, 0, 0)),
        out_shape=jax.ShapeDtypeStruct((L, OUT_F, B), jnp.float32),
    )(lngT, latT, sdneT, grids2, wpack, wst, consts)


@jax.jit
def kernel(lngs, lats, grid_id, SDNE_embedding, emb_table, W, b):
    idx = jnp.transpose(grid_id, (1, 0)).reshape(L, 8, PK)
    idx = jnp.transpose(idx, (0, 2, 1)).reshape(N).astype(jnp.int32)
    lngT = jnp.transpose(lngs, (1, 0)).reshape(L, 1, B)
    latT = jnp.transpose(lats, (1, 0)).reshape(L, 1, B)
    sdneT = jnp.transpose(SDNE_embedding, (1, 2, 0))

    grids = _gather_sc(emb_table, idx)
    grids2 = grids.reshape(N // 8, 128)

    wg = W[2:2 + EDIM]
    wpack = jax.scipy.linalg.block_diag(*([wg] * 8))
    wst = jnp.transpose(W[2 + EDIM:], (1, 0))
    consts = jnp.zeros((OUT_F, 128), jnp.float32)
    consts = consts.at[:, 0].set(W[0]).at[:, 1].set(W[1]).at[:, 2].set(b)

    outT = _dense_tc(lngT, latT, sdneT, grids2, wpack, wst, consts)
    return jnp.transpose(outT, (2, 0, 1))


# speedup vs baseline: 10.0402x; 1.1968x over previous
import functools

import jax
import jax.numpy as jnp
from jax import lax
from jax.experimental import pallas as pl
from jax.experimental.pallas import tpu as pltpu
from jax.experimental.pallas import tpu_sc as plsc

B, L = 4096, 200
N = B * L
EDIM = 16
OUT_F = 32
VOCAB = 65536

NC, NS = 2, 16
NW = NC * NS
PER_W = N // NW
CHUNK = 3200
NCHUNK = PER_W // CHUNK


def _gather_sc(table, idx):
    mesh = plsc.VectorSubcoreMesh(core_axis_name="c", subcore_axis_name="s")

    @functools.partial(
        pl.kernel,
        mesh=mesh,
        compiler_params=pltpu.CompilerParams(use_tc_tiling_on_sc=False),
        out_type=jax.ShapeDtypeStruct((N, EDIM), jnp.float32),
        scratch_types=[
            pltpu.VMEM((CHUNK,), jnp.int32),
            pltpu.VMEM((CHUNK,), jnp.int32),
            pltpu.VMEM((CHUNK, EDIM), jnp.float32),
            pltpu.VMEM((CHUNK, EDIM), jnp.float32),
            pltpu.SemaphoreType.DMA,
        ],
    )
    def k(table_hbm, idx_hbm, out_hbm, idx_v0, idx_v1, rows_v0, rows_v1,
          gsem):
        wid = lax.axis_index("s") * NC + lax.axis_index("c")
        base = wid * PER_W
        idx_bufs = (idx_v0, idx_v1)
        row_bufs = (rows_v0, rows_v1)

        def off(c):
            return base + c * CHUNK

        pltpu.sync_copy(idx_hbm.at[pl.ds(off(0), CHUNK)], idx_bufs[0])
        gathers = [pltpu.async_copy(table_hbm.at[idx_bufs[0]], row_bufs[0],
                                    gsem)]
        if NCHUNK > 1:
            pltpu.sync_copy(idx_hbm.at[pl.ds(off(1), CHUNK)], idx_bufs[1])
        for c in range(NCHUNK):
            gathers[c].wait()
            if c + 1 < NCHUNK:
                gathers.append(
                    pltpu.async_copy(table_hbm.at[idx_bufs[(c + 1) % 2]],
                                     row_bufs[(c + 1) % 2], gsem))
            pltpu.sync_copy(row_bufs[c % 2],
                            out_hbm.at[pl.ds(off(c), CHUNK)])
            if c + 2 < NCHUNK:
                pltpu.sync_copy(idx_hbm.at[pl.ds(off(c + 2), CHUNK)],
                                idx_bufs[c % 2])

    return k(table, idx)


PK = B // 8


LB = 2


def _dense_body(lng_ref, lat_ref, s_ref, g_ref, wpack_ref, wst_ref, c_ref,
                o_ref):
    f32 = jnp.float32
    for j in range(LB):
        S = s_ref[j]
        acc = lax.dot_general(wst_ref[...], S, (((1,), (0,)), ((), ())),
                              preferred_element_type=f32)
        R = lax.dot_general(g_ref[j * PK:(j + 1) * PK],
                            wpack_ref[...], (((1,), (0,)), ((), ())),
                            preferred_element_type=f32)
        RT = jnp.transpose(R, (1, 0))
        acc += jnp.concatenate(
            [RT[OUT_F * s:OUT_F * (s + 1), :] for s in range(8)], axis=1)
        acc += c_ref[:, 0:1] * lng_ref[j]
        acc += c_ref[:, 1:2] * lat_ref[j]
        acc += c_ref[:, 2:3]
        o_ref[j] = jnp.tanh(acc)


def _dense_tc(lngT, latT, sdneT, grids2, wpack, wst, consts):
    return pl.pallas_call(
        _dense_body,
        grid=(L // LB,),
        in_specs=[
            pl.BlockSpec((LB, 1, B), lambda i: (i, 0, 0)),
            pl.BlockSpec((LB, 1, B), lambda i: (i, 0, 0)),
            pl.BlockSpec((LB, EDIM, B), lambda i: (i, 0, 0)),
            pl.BlockSpec((LB * PK, 128), lambda i: (i, 0)),
            pl.BlockSpec((128, 8 * OUT_F), lambda i: (0, 0)),
            pl.BlockSpec((OUT_F, EDIM), lambda i: (0, 0)),
            pl.BlockSpec((OUT_F, 128), lambda i: (0, 0)),
        ],
        out_specs=pl.BlockSpec((LB, OUT_F, B), lambda i: (i, 0, 0)),
        out_shape=jax.ShapeDtypeStruct((L, OUT_F, B), jnp.float32),
    )(lngT, latT, sdneT, grids2, wpack, wst, consts)


@jax.jit
def kernel(lngs, lats, grid_id, SDNE_embedding, emb_table, W, b):
    idx = jnp.transpose(grid_id.reshape(8, PK, L), (2, 1, 0))
    idx = idx.reshape(N).astype(jnp.int32)
    lngT = jnp.transpose(lngs, (1, 0)).reshape(L, 1, B)
    latT = jnp.transpose(lats, (1, 0)).reshape(L, 1, B)
    sdneT = jnp.transpose(SDNE_embedding, (1, 2, 0))

    grids = _gather_sc(emb_table, idx)
    grids2 = grids.reshape(N // 8, 128)

    wg = W[2:2 + EDIM]
    wpack = jax.scipy.linalg.block_diag(*([wg] * 8))
    wst = jnp.transpose(W[2 + EDIM:], (1, 0))
    consts = jnp.zeros((OUT_F, 128), jnp.float32)
    consts = consts.at[:, 0].set(W[0]).at[:, 1].set(W[1]).at[:, 2].set(b)

    outT = _dense_tc(lngT, latT, sdneT, grids2, wpack, wst, consts)
    return jnp.transpose(outT, (2, 0, 1))


# speedup vs baseline: 11.8170x; 1.1770x over previous
import functools

import jax
import jax.numpy as jnp
from jax import lax
from jax.experimental import pallas as pl
from jax.experimental.pallas import tpu as pltpu
from jax.experimental.pallas import tpu_sc as plsc

B, L = 4096, 200
N = B * L
EDIM = 16
OUT_F = 32
VOCAB = 65536

NC, NS = 2, 16
NW = NC * NS
PER_W = N // NW
CHUNK = 3200
NCHUNK = PER_W // CHUNK


def _gather_sc(table, idx, pos):
    mesh = plsc.VectorSubcoreMesh(core_axis_name="c", subcore_axis_name="s")

    @functools.partial(
        pl.kernel,
        mesh=mesh,
        compiler_params=pltpu.CompilerParams(use_tc_tiling_on_sc=False),
        out_type=jax.ShapeDtypeStruct((N, EDIM), jnp.float32),
        scratch_types=[
            pltpu.VMEM((CHUNK,), jnp.int32),
            pltpu.VMEM((CHUNK,), jnp.int32),
            pltpu.VMEM((CHUNK,), jnp.int32),
            pltpu.VMEM((CHUNK,), jnp.int32),
            pltpu.VMEM((CHUNK, EDIM), jnp.float32),
            pltpu.VMEM((CHUNK, EDIM), jnp.float32),
            pltpu.SemaphoreType.DMA,
        ],
    )
    def k(table_hbm, idx_hbm, pos_hbm, out_hbm, idx_v0, idx_v1, pos_v0,
          pos_v1, rows_v0, rows_v1, gsem):
        wid = lax.axis_index("s") * NC + lax.axis_index("c")
        base = wid * PER_W
        idx_bufs = (idx_v0, idx_v1)
        pos_bufs = (pos_v0, pos_v1)
        row_bufs = (rows_v0, rows_v1)

        def off(c):
            return base + c * CHUNK

        pltpu.sync_copy(idx_hbm.at[pl.ds(off(0), CHUNK)], idx_bufs[0])
        pltpu.sync_copy(pos_hbm.at[pl.ds(off(0), CHUNK)], pos_bufs[0])
        gathers = [pltpu.async_copy(table_hbm.at[idx_bufs[0]], row_bufs[0],
                                    gsem)]
        if NCHUNK > 1:
            pltpu.sync_copy(idx_hbm.at[pl.ds(off(1), CHUNK)], idx_bufs[1])
            pltpu.sync_copy(pos_hbm.at[pl.ds(off(1), CHUNK)], pos_bufs[1])
        for c in range(NCHUNK):
            gathers[c].wait()
            if c + 1 < NCHUNK:
                gathers.append(
                    pltpu.async_copy(table_hbm.at[idx_bufs[(c + 1) % 2]],
                                     row_bufs[(c + 1) % 2], gsem))
            pltpu.sync_copy(row_bufs[c % 2], out_hbm.at[pos_bufs[c % 2]])
            if c + 2 < NCHUNK:
                pltpu.sync_copy(idx_hbm.at[pl.ds(off(c + 2), CHUNK)],
                                idx_bufs[c % 2])
                pltpu.sync_copy(pos_hbm.at[pl.ds(off(c + 2), CHUNK)],
                                pos_bufs[c % 2])

    return k(table, idx, pos)


PK = B // 8


LB = 2


def _dense_body(lng_ref, lat_ref, s_ref, g_ref, wpack_ref, wst_ref, c_ref,
                o_ref):
    f32 = jnp.float32
    for j in range(LB):
        S = s_ref[j]
        acc = lax.dot_general(wst_ref[...], S, (((1,), (0,)), ((), ())),
                              preferred_element_type=f32)
        R = lax.dot_general(g_ref[j * PK:(j + 1) * PK],
                            wpack_ref[...], (((1,), (0,)), ((), ())),
                            preferred_element_type=f32)
        RT = jnp.transpose(R, (1, 0))
        acc += jnp.concatenate(
            [RT[OUT_F * s:OUT_F * (s + 1), :] for s in range(8)], axis=1)
        acc += c_ref[:, 0:1] * lng_ref[j]
        acc += c_ref[:, 1:2] * lat_ref[j]
        acc += c_ref[:, 2:3]
        o_ref[j] = jnp.tanh(acc)


def _dense_tc(lngT, latT, sdneT, grids2, wpack, wst, consts):
    return pl.pallas_call(
        _dense_body,
        grid=(L // LB,),
        in_specs=[
            pl.BlockSpec((LB, 1, B), lambda i: (i, 0, 0)),
            pl.BlockSpec((LB, 1, B), lambda i: (i, 0, 0)),
            pl.BlockSpec((LB, EDIM, B), lambda i: (i, 0, 0)),
            pl.BlockSpec((LB * PK, 128), lambda i: (i, 0)),
            pl.BlockSpec((128, 8 * OUT_F), lambda i: (0, 0)),
            pl.BlockSpec((OUT_F, EDIM), lambda i: (0, 0)),
            pl.BlockSpec((OUT_F, 128), lambda i: (0, 0)),
        ],
        out_specs=pl.BlockSpec((LB, OUT_F, B), lambda i: (i, 0, 0)),
        out_shape=jax.ShapeDtypeStruct((L, OUT_F, B), jnp.float32),
    )(lngT, latT, sdneT, grids2, wpack, wst, consts)


@jax.jit
def kernel(lngs, lats, grid_id, SDNE_embedding, emb_table, W, b):
    idx = jnp.transpose(grid_id, (1, 0)).reshape(N).astype(jnp.int32)
    t = jnp.arange(N, dtype=jnp.int32)
    tb = t % B
    pos = (t // B) * B + (tb % PK) * 8 + tb // PK
    lngT = jnp.transpose(lngs, (1, 0)).reshape(L, 1, B)
    latT = jnp.transpose(lats, (1, 0)).reshape(L, 1, B)
    sdneT = jnp.transpose(SDNE_embedding, (1, 2, 0))

    grids = _gather_sc(emb_table, idx, pos)
    grids2 = grids.reshape(N // 8, 128)

    wg = W[2:2 + EDIM]
    wpack = jax.scipy.linalg.block_diag(*([wg] * 8))
    wst = jnp.transpose(W[2 + EDIM:], (1, 0))
    consts = jnp.zeros((OUT_F, 128), jnp.float32)
    consts = consts.at[:, 0].set(W[0]).at[:, 1].set(W[1]).at[:, 2].set(b)

    outT = _dense_tc(lngT, latT, sdneT, grids2, wpack, wst, consts)
    return jnp.transpose(outT, (2, 0, 1))


# speedup vs baseline: 13.0661x; 1.1057x over previous
import functools

import jax
import jax.numpy as jnp
from jax import lax
from jax.experimental import pallas as pl
from jax.experimental.pallas import tpu as pltpu
from jax.experimental.pallas import tpu_sc as plsc

B, L = 4096, 200
N = B * L
EDIM = 16
OUT_F = 32
VOCAB = 65536

NC, NS = 2, 16
NW = NC * NS
PER_W = N // NW
CHUNK = 3200
NCHUNK = PER_W // CHUNK


def _gather_sc(table, idx, pos):
    mesh = plsc.VectorSubcoreMesh(core_axis_name="c", subcore_axis_name="s")

    @functools.partial(
        pl.kernel,
        mesh=mesh,
        compiler_params=pltpu.CompilerParams(use_tc_tiling_on_sc=False),
        out_type=jax.ShapeDtypeStruct((N, EDIM), jnp.float32),
        scratch_types=[
            pltpu.VMEM((CHUNK,), jnp.int32),
            pltpu.VMEM((CHUNK,), jnp.int32),
            pltpu.VMEM((CHUNK,), jnp.int32),
            pltpu.VMEM((CHUNK,), jnp.int32),
            pltpu.VMEM((CHUNK, EDIM), jnp.float32),
            pltpu.VMEM((CHUNK, EDIM), jnp.float32),
            pltpu.SemaphoreType.DMA,
        ],
    )
    def k(table_hbm, idx_hbm, pos_hbm, out_hbm, idx_v0, idx_v1, pos_v0,
          pos_v1, rows_v0, rows_v1, gsem):
        wid = lax.axis_index("s") * NC + lax.axis_index("c")
        base = wid * PER_W
        idx_bufs = (idx_v0, idx_v1)
        pos_bufs = (pos_v0, pos_v1)
        row_bufs = (rows_v0, rows_v1)

        def off(c):
            return base + c * CHUNK

        pltpu.sync_copy(idx_hbm.at[pl.ds(off(0), CHUNK)], idx_bufs[0])
        pltpu.sync_copy(pos_hbm.at[pl.ds(off(0), CHUNK)], pos_bufs[0])
        gathers = [pltpu.async_copy(table_hbm.at[idx_bufs[0]], row_bufs[0],
                                    gsem)]
        if NCHUNK > 1:
            pltpu.sync_copy(idx_hbm.at[pl.ds(off(1), CHUNK)], idx_bufs[1])
            pltpu.sync_copy(pos_hbm.at[pl.ds(off(1), CHUNK)], pos_bufs[1])
        for c in range(NCHUNK):
            gathers[c].wait()
            if c + 1 < NCHUNK:
                gathers.append(
                    pltpu.async_copy(table_hbm.at[idx_bufs[(c + 1) % 2]],
                                     row_bufs[(c + 1) % 2], gsem))
            pltpu.sync_copy(row_bufs[c % 2], out_hbm.at[pos_bufs[c % 2]])
            if c + 2 < NCHUNK:
                pltpu.sync_copy(idx_hbm.at[pl.ds(off(c + 2), CHUNK)],
                                idx_bufs[c % 2])
                pltpu.sync_copy(pos_hbm.at[pl.ds(off(c + 2), CHUNK)],
                                pos_bufs[c % 2])

    return k(table, idx, pos)


PK = B // 8


LB = 4


def _dense_body(lng_ref, lat_ref, s_ref, g_ref, wpack_ref, wst_ref, c_ref,
                o_ref):
    f32 = jnp.float32
    for j in range(LB):
        S = s_ref[j]
        acc = lax.dot_general(wst_ref[...], S, (((1,), (0,)), ((), ())),
                              preferred_element_type=f32)
        R = lax.dot_general(g_ref[j * PK:(j + 1) * PK],
                            wpack_ref[...], (((1,), (0,)), ((), ())),
                            preferred_element_type=f32)
        RT = jnp.transpose(R, (1, 0))
        acc += jnp.concatenate(
            [RT[OUT_F * s:OUT_F * (s + 1), :] for s in range(8)], axis=1)
        acc += c_ref[:, 0:1] * lng_ref[j]
        acc += c_ref[:, 1:2] * lat_ref[j]
        acc += c_ref[:, 2:3]
        o_ref[j] = jnp.tanh(acc)


def _dense_tc(lngT, latT, sdneT, grids2, wpack, wst, consts):
    return pl.pallas_call(
        _dense_body,
        grid=(L // LB,),
        in_specs=[
            pl.BlockSpec((LB, 1, B), lambda i: (i, 0, 0)),
            pl.BlockSpec((LB, 1, B), lambda i: (i, 0, 0)),
            pl.BlockSpec((LB, EDIM, B), lambda i: (i, 0, 0)),
            pl.BlockSpec((LB * PK, 128), lambda i: (i, 0)),
            pl.BlockSpec((128, 8 * OUT_F), lambda i: (0, 0)),
            pl.BlockSpec((OUT_F, EDIM), lambda i: (0, 0)),
            pl.BlockSpec((OUT_F, 128), lambda i: (0, 0)),
        ],
        out_specs=pl.BlockSpec((LB, OUT_F, B), lambda i: (i, 0, 0)),
        out_shape=jax.ShapeDtypeStruct((L, OUT_F, B), jnp.float32),
    )(lngT, latT, sdneT, grids2, wpack, wst, consts)


@jax.jit
def kernel(lngs, lats, grid_id, SDNE_embedding, emb_table, W, b):
    idx = jnp.transpose(grid_id, (1, 0)).reshape(N).astype(jnp.int32)
    t = jnp.arange(N, dtype=jnp.int32)
    tb = t % B
    pos = (t // B) * B + (tb % PK) * 8 + tb // PK
    lngT = jnp.transpose(lngs, (1, 0)).reshape(L, 1, B)
    latT = jnp.transpose(lats, (1, 0)).reshape(L, 1, B)
    sdneT = jnp.transpose(SDNE_embedding, (1, 2, 0))

    grids = _gather_sc(emb_table, idx, pos)
    grids2 = grids.reshape(N // 8, 128)

    wg = W[2:2 + EDIM]
    wpack = jax.scipy.linalg.block_diag(*([wg] * 8))
    wst = jnp.transpose(W[2 + EDIM:], (1, 0))
    consts = jnp.zeros((OUT_F, 128), jnp.float32)
    consts = consts.at[:, 0].set(W[0]).at[:, 1].set(W[1]).at[:, 2].set(b)

    outT = _dense_tc(lngT, latT, sdneT, grids2, wpack, wst, consts)
    return jnp.transpose(outT, (2, 0, 1))


# speedup vs baseline: 13.8462x; 1.0597x over previous
import functools

import jax
import jax.numpy as jnp
from jax import lax
from jax.experimental import pallas as pl
from jax.experimental.pallas import tpu as pltpu
from jax.experimental.pallas import tpu_sc as plsc

B, L = 4096, 200
N = B * L
EDIM = 16
OUT_F = 32
VOCAB = 65536

NC, NS = 2, 16
NW = NC * NS
PER_W = N // NW
CHUNK = 3200
NCHUNK = PER_W // CHUNK


def _gather_sc(table, idx, pos):
    mesh = plsc.VectorSubcoreMesh(core_axis_name="c", subcore_axis_name="s")

    @functools.partial(
        pl.kernel,
        mesh=mesh,
        compiler_params=pltpu.CompilerParams(use_tc_tiling_on_sc=False),
        out_type=jax.ShapeDtypeStruct((N, EDIM), jnp.float32),
        scratch_types=[
            pltpu.VMEM((CHUNK,), jnp.int32),
            pltpu.VMEM((CHUNK,), jnp.int32),
            pltpu.VMEM((CHUNK,), jnp.int32),
            pltpu.VMEM((CHUNK,), jnp.int32),
            pltpu.VMEM((CHUNK, EDIM), jnp.float32),
            pltpu.VMEM((CHUNK, EDIM), jnp.float32),
            pltpu.SemaphoreType.DMA,
        ],
    )
    def k(table_hbm, idx_hbm, pos_hbm, out_hbm, idx_v0, idx_v1, pos_v0,
          pos_v1, rows_v0, rows_v1, gsem):
        wid = lax.axis_index("s") * NC + lax.axis_index("c")
        base = wid * PER_W
        idx_bufs = (idx_v0, idx_v1)
        pos_bufs = (pos_v0, pos_v1)
        row_bufs = (rows_v0, rows_v1)

        def off(c):
            return base + c * CHUNK

        pltpu.sync_copy(idx_hbm.at[pl.ds(off(0), CHUNK)], idx_bufs[0])
        pltpu.sync_copy(pos_hbm.at[pl.ds(off(0), CHUNK)], pos_bufs[0])
        gathers = [pltpu.async_copy(table_hbm.at[idx_bufs[0]], row_bufs[0],
                                    gsem)]
        if NCHUNK > 1:
            pltpu.sync_copy(idx_hbm.at[pl.ds(off(1), CHUNK)], idx_bufs[1])
            pltpu.sync_copy(pos_hbm.at[pl.ds(off(1), CHUNK)], pos_bufs[1])
        for c in range(NCHUNK):
            gathers[c].wait()
            if c + 1 < NCHUNK:
                gathers.append(
                    pltpu.async_copy(table_hbm.at[idx_bufs[(c + 1) % 2]],
                                     row_bufs[(c + 1) % 2], gsem))
            pltpu.sync_copy(row_bufs[c % 2], out_hbm.at[pos_bufs[c % 2]])
            if c + 2 < NCHUNK:
                pltpu.sync_copy(idx_hbm.at[pl.ds(off(c + 2), CHUNK)],
                                idx_bufs[c % 2])
                pltpu.sync_copy(pos_hbm.at[pl.ds(off(c + 2), CHUNK)],
                                pos_bufs[c % 2])

    return k(table, idx, pos)


PK = B // 8


LB = 8


def _dense_body(lng_ref, lat_ref, s_ref, g_ref, wpack_ref, wst_ref, c_ref,
                o_ref):
    f32 = jnp.float32
    for j in range(LB):
        S = s_ref[j]
        acc = lax.dot_general(wst_ref[...], S, (((1,), (0,)), ((), ())),
                              preferred_element_type=f32)
        R = lax.dot_general(g_ref[j * PK:(j + 1) * PK],
                            wpack_ref[...], (((1,), (0,)), ((), ())),
                            preferred_element_type=f32)
        RT = jnp.transpose(R, (1, 0))
        acc += jnp.concatenate(
            [RT[OUT_F * s:OUT_F * (s + 1), :] for s in range(8)], axis=1)
        acc += c_ref[:, 0:1] * lng_ref[j]
        acc += c_ref[:, 1:2] * lat_ref[j]
        acc += c_ref[:, 2:3]
        o_ref[j] = jnp.tanh(acc)


def _dense_tc(lngT, latT, sdneT, grids2, wpack, wst, consts):
    return pl.pallas_call(
        _dense_body,
        grid=(L // LB,),
        in_specs=[
            pl.BlockSpec((LB, 1, B), lambda i: (i, 0, 0)),
            pl.BlockSpec((LB, 1, B), lambda i: (i, 0, 0)),
            pl.BlockSpec((LB, EDIM, B), lambda i: (i, 0, 0)),
            pl.BlockSpec((LB * PK, 128), lambda i: (i, 0)),
            pl.BlockSpec((128, 8 * OUT_F), lambda i: (0, 0)),
            pl.BlockSpec((OUT_F, EDIM), lambda i: (0, 0)),
            pl.BlockSpec((OUT_F, 128), lambda i: (0, 0)),
        ],
        out_specs=pl.BlockSpec((LB, OUT_F, B), lambda i: (i, 0, 0)),
        out_shape=jax.ShapeDtypeStruct((L, OUT_F, B), jnp.float32),
    )(lngT, latT, sdneT, grids2, wpack, wst, consts)


@jax.jit
def kernel(lngs, lats, grid_id, SDNE_embedding, emb_table, W, b):
    idx = jnp.transpose(grid_id, (1, 0)).reshape(N).astype(jnp.int32)
    t = jnp.arange(N, dtype=jnp.int32)
    tb = t % B
    pos = (t // B) * B + (tb % PK) * 8 + tb // PK
    lngT = jnp.transpose(lngs, (1, 0)).reshape(L, 1, B)
    latT = jnp.transpose(lats, (1, 0)).reshape(L, 1, B)
    sdneT = jnp.transpose(SDNE_embedding, (1, 2, 0))

    grids = _gather_sc(emb_table, idx, pos)
    grids2 = grids.reshape(N // 8, 128)

    wg = W[2:2 + EDIM]
    wpack = jax.scipy.linalg.block_diag(*([wg] * 8))
    wst = jnp.transpose(W[2 + EDIM:], (1, 0))
    consts = jnp.zeros((OUT_F, 128), jnp.float32)
    consts = consts.at[:, 0].set(W[0]).at[:, 1].set(W[1]).at[:, 2].set(b)

    outT = _dense_tc(lngT, latT, sdneT, grids2, wpack, wst, consts)
    return jnp.transpose(outT, (2, 0, 1))


# speedup vs baseline: 14.0804x; 1.0169x over previous
import functools

import jax
import jax.numpy as jnp
from jax import lax
from jax.experimental import pallas as pl
from jax.experimental.pallas import tpu as pltpu
from jax.experimental.pallas import tpu_sc as plsc

B, L = 4096, 200
N = B * L
EDIM = 16
OUT_F = 32
VOCAB = 65536

NC, NS = 2, 16
NW = NC * NS
PER_W = N // NW
CHUNK = 3200
NCHUNK = PER_W // CHUNK


def _gather_sc(table, idx, pos):
    mesh = plsc.VectorSubcoreMesh(core_axis_name="c", subcore_axis_name="s")

    @functools.partial(
        pl.kernel,
        mesh=mesh,
        compiler_params=pltpu.CompilerParams(use_tc_tiling_on_sc=False),
        out_type=jax.ShapeDtypeStruct((N, EDIM), jnp.float32),
        scratch_types=[
            pltpu.VMEM((CHUNK,), jnp.int32),
            pltpu.VMEM((CHUNK,), jnp.int32),
            pltpu.VMEM((CHUNK,), jnp.int32),
            pltpu.VMEM((CHUNK,), jnp.int32),
            pltpu.VMEM((CHUNK, EDIM), jnp.float32),
            pltpu.VMEM((CHUNK, EDIM), jnp.float32),
            pltpu.SemaphoreType.DMA,
        ],
    )
    def k(table_hbm, idx_hbm, pos_hbm, out_hbm, idx_v0, idx_v1, pos_v0,
          pos_v1, rows_v0, rows_v1, gsem):
        wid = lax.axis_index("s") * NC + lax.axis_index("c")
        base = wid * PER_W
        idx_bufs = (idx_v0, idx_v1)
        pos_bufs = (pos_v0, pos_v1)
        row_bufs = (rows_v0, rows_v1)

        def off(c):
            return base + c * CHUNK

        pltpu.sync_copy(idx_hbm.at[pl.ds(off(0), CHUNK)], idx_bufs[0])
        pltpu.sync_copy(pos_hbm.at[pl.ds(off(0), CHUNK)], pos_bufs[0])
        gathers = [pltpu.async_copy(table_hbm.at[idx_bufs[0]], row_bufs[0],
                                    gsem)]
        if NCHUNK > 1:
            pltpu.sync_copy(idx_hbm.at[pl.ds(off(1), CHUNK)], idx_bufs[1])
            pltpu.sync_copy(pos_hbm.at[pl.ds(off(1), CHUNK)], pos_bufs[1])
        for c in range(NCHUNK):
            gathers[c].wait()
            if c + 1 < NCHUNK:
                gathers.append(
                    pltpu.async_copy(table_hbm.at[idx_bufs[(c + 1) % 2]],
                                     row_bufs[(c + 1) % 2], gsem))
            pltpu.sync_copy(row_bufs[c % 2], out_hbm.at[pos_bufs[c % 2]])
            if c + 2 < NCHUNK:
                pltpu.sync_copy(idx_hbm.at[pl.ds(off(c + 2), CHUNK)],
                                idx_bufs[c % 2])
                pltpu.sync_copy(pos_hbm.at[pl.ds(off(c + 2), CHUNK)],
                                pos_bufs[c % 2])

    return k(table, idx, pos)


PK = B // 8


LB = 10


def _dense_body(lng_ref, lat_ref, s_ref, g_ref, wpack_ref, wst_ref, c_ref,
                o_ref):
    f32 = jnp.float32
    for j in range(LB):
        S = s_ref[j]
        acc = lax.dot_general(wst_ref[...], S, (((1,), (0,)), ((), ())),
                              preferred_element_type=f32)
        R = lax.dot_general(g_ref[j * PK:(j + 1) * PK],
                            wpack_ref[...], (((1,), (0,)), ((), ())),
                            preferred_element_type=f32)
        RT = jnp.transpose(R, (1, 0))
        acc += jnp.concatenate(
            [RT[OUT_F * s:OUT_F * (s + 1), :] for s in range(8)], axis=1)
        acc += c_ref[:, 0:1] * lng_ref[j]
        acc += c_ref[:, 1:2] * lat_ref[j]
        acc += c_ref[:, 2:3]
        o_ref[j] = jnp.tanh(acc)


def _dense_tc(lngT, latT, sdneT, grids2, wpack, wst, consts):
    return pl.pallas_call(
        _dense_body,
        grid=(L // LB,),
        in_specs=[
            pl.BlockSpec((LB, 1, B), lambda i: (i, 0, 0)),
            pl.BlockSpec((LB, 1, B), lambda i: (i, 0, 0)),
            pl.BlockSpec((LB, EDIM, B), lambda i: (i, 0, 0)),
            pl.BlockSpec((LB * PK, 128), lambda i: (i, 0)),
            pl.BlockSpec((128, 8 * OUT_F), lambda i: (0, 0)),
            pl.BlockSpec((OUT_F, EDIM), lambda i: (0, 0)),
            pl.BlockSpec((OUT_F, 128), lambda i: (0, 0)),
        ],
        out_specs=pl.BlockSpec((LB, OUT_F, B), lambda i: (i, 0, 0)),
        out_shape=jax.ShapeDtypeStruct((L, OUT_F, B), jnp.float32),
    )(lngT, latT, sdneT, grids2, wpack, wst, consts)


@jax.jit
def kernel(lngs, lats, grid_id, SDNE_embedding, emb_table, W, b):
    idx = jnp.transpose(grid_id, (1, 0)).reshape(N).astype(jnp.int32)
    t = jnp.arange(N, dtype=jnp.int32)
    tb = t % B
    pos = (t // B) * B + (tb % PK) * 8 + tb // PK
    lngT = jnp.transpose(lngs, (1, 0)).reshape(L, 1, B)
    latT = jnp.transpose(lats, (1, 0)).reshape(L, 1, B)
    sdneT = jnp.transpose(SDNE_embedding, (1, 2, 0))

    grids = _gather_sc(emb_table, idx, pos)
    grids2 = grids.reshape(N // 8, 128)

    wg = W[2:2 + EDIM]
    wpack = jax.scipy.linalg.block_diag(*([wg] * 8))
    wst = jnp.transpose(W[2 + EDIM:], (1, 0))
    consts = jnp.zeros((OUT_F, 128), jnp.float32)
    consts = consts.at[:, 0].set(W[0]).at[:, 1].set(W[1]).at[:, 2].set(b)

    outT = _dense_tc(lngT, latT, sdneT, grids2, wpack, wst, consts)
    return jnp.transpose(outT, (2, 0, 1))
